# baseline (speedup 1.0000x reference)
import functools

import jax
import jax.numpy as jnp
from jax import lax
from jax.experimental import pallas as pl
from jax.experimental.pallas import tpu as pltpu
from jax.experimental.pallas import tpu_sc as plsc

_N = 10000
_G = 8
_S = _N // _G
_SP = 1280
_NP = _G * _SP
_E = 160000
_HID = 256
_KP = _S // 2

_NTC = 16
_RPT = _NP // _NTC


def _pad_rows(a):
    f = a.shape[1]
    return jnp.pad(a.reshape(_G, _S, f), ((0, 0), (0, _SP - _S), (0, 0))
                   ).reshape(_NP, f)


def _sc_agg_stream(ys, src, dst, nchunks, ch, src_off, edge_base, init_off):
    mesh = plsc.VectorSubcoreMesh(core_axis_name="c", subcore_axis_name="s")

    @functools.partial(
        pl.kernel,
        out_type=jax.ShapeDtypeStruct((2 * _NP, 128), jnp.float32),
        mesh=mesh,
        scratch_types=[
            pltpu.VMEM((ch,), jnp.int32), pltpu.VMEM((ch,), jnp.int32),
            pltpu.VMEM((ch,), jnp.int32), pltpu.VMEM((ch,), jnp.int32),
            pltpu.VMEM((ch, 128), jnp.float32),
            pltpu.VMEM((ch, 128), jnp.float32),
            pltpu.VMEM_SHARED((_NP, 128), jnp.float32),
            pltpu.SemaphoreType.DMA, pltpu.SemaphoreType.DMA,
            pltpu.SemaphoreType.DMA, pltpu.SemaphoreType.DMA,
        ],
    )
    def k(ys_ref, src_ref, dst_ref, out_ref, sidx0, sidx1, didx0, didx1,
          rows0, rows1, acc, gsem0, gsem1, ssem0, ssem1):
        c = lax.axis_index("c")
        s = lax.axis_index("s")
        r0 = s * _RPT
        pltpu.sync_copy(ys_ref.at[pl.ds(init_off(c) + r0, _RPT)],
                        acc.at[pl.ds(r0, _RPT)])
        plsc.subcore_barrier()
        base = edge_base(c, s)
        soff = src_off(c)
        sidx = [sidx0, sidx1]
        didx = [didx0, didx1]
        rows = [rows0, rows1]
        gsem = [gsem0, gsem1]
        ssem = [ssem0, ssem1]
        gath = [None, None]
        scat = [None, None]
        e0 = base
        pltpu.sync_copy(src_ref.at[pl.ds(soff + e0, ch)], sidx[0])
        pltpu.sync_copy(dst_ref.at[pl.ds(e0, ch)], didx[0])
        gath[0] = pltpu.async_copy(ys_ref.at[sidx[0]], rows[0], gsem[0])
        for j in range(nchunks):
            b = j & 1
            nb = b ^ 1
            if j + 1 < nchunks:
                if j >= 1:
                    scat[nb].wait()
                e1 = base + (j + 1) * ch
                pltpu.sync_copy(src_ref.at[pl.ds(soff + e1, ch)], sidx[nb])
                pltpu.sync_copy(dst_ref.at[pl.ds(e1, ch)], didx[nb])
                gath[nb] = pltpu.async_copy(ys_ref.at[sidx[nb]], rows[nb],
                                            gsem[nb])
            gath[b].wait()
            scat[b] = pltpu.async_copy(rows[b], acc.at[didx[b]], ssem[b],
                                       add=True)
        scat[(nchunks - 1) & 1].wait()
        if nchunks >= 2:
            scat[(nchunks - 2) & 1].wait()
        plsc.subcore_barrier()
        pltpu.sync_copy(acc.at[pl.ds(r0, _RPT)],
                        out_ref.at[pl.ds(c * _NP + r0, _RPT)])

    return k(ys, src, dst)


_CH = 184
_EPTC = _CH * 55
_EPTE = _CH * 28
_ECP = _NTC * _EPTC
_EEP = 2 * _NTC * _EPTE


def _sc_agg_cols(ys2, src2c, dstc):
    return _sc_agg_stream(
        ys2, src2c, dstc, nchunks=55, ch=_CH,
        src_off=lambda c: c * _ECP,
        edge_base=lambda c, s: s * _EPTC,
        init_off=lambda c: c * _NP)


def _sc_agg_edges(ys, srce, dste):
    return _sc_agg_stream(
        ys, srce, dste, nchunks=28, ch=_CH,
        src_off=lambda c: 0,
        edge_base=lambda c, s: (c * _NTC + s) * _EPTE,
        init_off=lambda c: 0)


def _pad_tiles(a, ntiles, ept_pad):
    per = _E // ntiles
    return jnp.pad(a.reshape(ntiles, per), ((0, 0), (0, ept_pad - per)),
                   constant_values=_S).reshape(-1)


_EPAD = 160256
_EPT1 = _EPAD // 32


def _sc_agg_scalar(yv, srcPp, dstPp):
    mesh = plsc.VectorSubcoreMesh(core_axis_name="c", subcore_axis_name="s")

    @functools.partial(
        pl.kernel,
        out_type=jax.ShapeDtypeStruct((2 * _NP,), jnp.float32),
        mesh=mesh,
        compiler_params=pltpu.CompilerParams(needs_layout_passes=False),
        scratch_types=[
            pltpu.VMEM((_EPT1,), jnp.int32),
            pltpu.VMEM((_EPT1,), jnp.int32),
            pltpu.VMEM((_NP,), jnp.float32),
            pltpu.VMEM((_NP,), jnp.float32),
            pltpu.VMEM((_NTC, _RPT), jnp.float32),
            pltpu.VMEM((_RPT,), jnp.float32),
            pltpu.VMEM_SHARED((_NTC, _NP), jnp.float32),
            pltpu.SemaphoreType.DMA,
        ],
    )
    def k(y_ref, src_ref, dst_ref, out_ref, sidx, didx, yv_, acc, red, res,
          shared, sem):
        c = lax.axis_index("c")
        s = lax.axis_index("s")
        w = c * _NTC + s
        pltpu.sync_copy(y_ref, yv_)
        pltpu.sync_copy(src_ref.at[pl.ds(w * _EPT1, _EPT1)], sidx)
        pltpu.sync_copy(dst_ref.at[pl.ds(w * _EPT1, _EPT1)], didx)

        def zbody(i, carry):
            acc[pl.ds(i * 16, 16)] = jnp.zeros((16,), jnp.float32)
            return carry

        lax.fori_loop(0, _NP // 16, zbody, 0)

        def ebody(i, carry):
            s16 = sidx[pl.ds(i * 16, 16)]
            d16 = didx[pl.ds(i * 16, 16)]
            vals = plsc.load_gather(yv_, [s16])
            plsc.addupdate_scatter(acc, [d16], vals)
            return carry

        lax.fori_loop(0, _EPT1 // 16, ebody, 0)
        pltpu.sync_copy(acc, shared.at[s])
        plsc.subcore_barrier()
        lo = s * _RPT
        pltpu.sync_copy(shared.at[:, pl.ds(lo, _RPT)], red)

        def rbody(j, carry):
            v = red[0, pl.ds(j * 16, 16)]
            for t in range(1, _NTC):
                v = v + red[t, pl.ds(j * 16, 16)]
            res[pl.ds(j * 16, 16)] = v
            return carry

        lax.fori_loop(0, _RPT // 16, rbody, 0)
        pltpu.sync_copy(res, out_ref.at[pl.ds(c * _NP + lo, _RPT)])

    return k(yv, srcPp, dstPp)


def _rs(f=1):
    return pl.BlockSpec((_SP, f), lambda g: (g, 0))


def _rs2(f=1):
    return pl.BlockSpec((_SP, f), lambda g, h: (g, 0))


def _fs(shape):
    nd = len(shape)
    return pl.BlockSpec(shape, lambda g: (0,) * nd)


def _fs2(shape):
    nd = len(shape)
    return pl.BlockSpec(shape, lambda g, h: (0,) * nd)


def _t0_body(da_ref, db_ref, x_ref, dinv_ref, x1_ref):
    dinv = jax.lax.rsqrt(da_ref[...] + db_ref[...] + 1.0)
    dinv_ref[...] = dinv
    x1_ref[...] = x_ref[...] * dinv


def _t0(degV, xP):
    return pl.pallas_call(
        _t0_body,
        grid=(_G,),
        in_specs=[pl.BlockSpec((_SP, 1), lambda g: (g, 0)),
                  pl.BlockSpec((_SP, 1), lambda g: (_G + g, 0)),
                  _rs(128)],
        out_specs=[_rs(), _rs(128)],
        out_shape=[jax.ShapeDtypeStruct((_NP, 1), jnp.float32),
                   jax.ShapeDtypeStruct((_NP, 128), jnp.float32)],
    )(degV, degV, xP)


def _t0b_body(pa_ref, pb_ref, x1_ref, dinv_ref, w0_ref, b0_ref, w1h_ref,
              h0_ref, y1_ref):
    agg = dinv_ref[...] * (pa_ref[...] + pb_ref[...] - x1_ref[...])
    h0 = jnp.maximum(
        jnp.dot(agg, w0_ref[...], preferred_element_type=jnp.float32)
        + b0_ref[...], 0.0)
    h0_ref[...] = h0
    y1_ref[...] = jnp.dot(h0, w1h_ref[...], preferred_element_type=jnp.float32
                          ) * dinv_ref[...]


def _t0b(P0v, x1, dinvP, W0, b0, W1):
    return pl.pallas_call(
        _t0b_body,
        grid=(_G, 2),
        in_specs=[pl.BlockSpec((_SP, 128), lambda g, h: (g, 0)),
                  pl.BlockSpec((_SP, 128), lambda g, h: (_G + g, 0)),
                  _rs2(128), _rs2(),
                  _fs2((128, _HID)), _fs2((1, _HID)),
                  pl.BlockSpec((_HID, 128), lambda g, h: (0, h))],
        out_specs=[pl.BlockSpec((_SP, _HID), lambda g, h: (g, 0)),
                   pl.BlockSpec((_SP, 128), lambda g, h: (h * _G + g, 0))],
        out_shape=[jax.ShapeDtypeStruct((_NP, _HID), jnp.float32),
                   jax.ShapeDtypeStruct((2 * _NP, 128), jnp.float32)],
    )(P0v, P0v, x1, dinvP, W0, b0.reshape(1, _HID), W1)


def _glob_body(h0_ref, wfc_ref, bfc_ref, w2b_ref, gb_ref):
    h = h0_ref[...]
    rows = jax.lax.broadcasted_iota(jnp.int32, h.shape, 1)
    h = jnp.where(rows < _S, h, -1e30)
    glob = jnp.max(h, axis=1)
    gi = jnp.dot(glob, wfc_ref[...], preferred_element_type=jnp.float32
                 ) + bfc_ref[...]
    gb_ref[...] = jnp.dot(gi, w2b_ref[...], preferred_element_type=jnp.float32)


def _glob(h0, Wfc, bfc, W2b):
    return pl.pallas_call(
        _glob_body,
        grid=(1,),
        in_specs=[_fs((_G, _SP, _HID)), _fs((_HID, _HID)),
                  _fs((1, _HID)), _fs((_HID, _HID))],
        out_specs=_fs((_G, _HID)),
        out_shape=jax.ShapeDtypeStruct((_G, _HID), jnp.float32),
    )(h0.reshape(_G, _SP, _HID), Wfc, bfc.reshape(1, _HID), W2b)


def _mid_body(pa_ref, pb_ref, dinv_ref, b_ref, wh_ref, out_ref):
    agg = dinv_ref[...] * jnp.concatenate([pa_ref[...], pb_ref[...]], axis=1)
    h = jnp.maximum(agg + b_ref[...], 0.0)
    out_ref[...] = jnp.dot(h, wh_ref[...], preferred_element_type=jnp.float32
                           ) * dinv_ref[...]


def _mid(Pv, dinvP, b, W):
    return pl.pallas_call(
        _mid_body,
        grid=(_G, 2),
        in_specs=[pl.BlockSpec((_SP, 128), lambda g, h: (g, 0)),
                  pl.BlockSpec((_SP, 128), lambda g, h: (_G + g, 0)),
                  _rs2(), _fs2((1, _HID)),
                  pl.BlockSpec((_HID, 128), lambda g, h: (0, h))],
        out_specs=pl.BlockSpec((_SP, 128), lambda g, h: (h * _G + g, 0)),
        out_shape=jax.ShapeDtypeStruct((2 * _NP, 128), jnp.float32),
    )(Pv, Pv, dinvP, b.reshape(1, _HID), W)


def _t3_body(pa_ref, pb_ref, dinv_ref, b_ref, wh_ref, gb_ref, out_ref):
    agg = dinv_ref[...] * jnp.concatenate([pa_ref[...], pb_ref[...]], axis=1)
    h = jnp.maximum(agg + b_ref[...], 0.0)
    y3 = jnp.dot(h, wh_ref[...], preferred_element_type=jnp.float32
                 ) + gb_ref[0]
    out_ref[...] = y3 * dinv_ref[...]


def _t3(P2v, dinvP, b1, W2a, gb):
    return pl.pallas_call(
        _t3_body,
        grid=(_G, 2),
        in_specs=[pl.BlockSpec((_SP, 128), lambda g, h: (g, 0)),
                  pl.BlockSpec((_SP, 128), lambda g, h: (_G + g, 0)),
                  _rs2(), _fs2((1, _HID)),
                  pl.BlockSpec((_HID, 128), lambda g, h: (0, h)),
                  pl.BlockSpec((1, 1, 128), lambda g, h: (g, 0, h))],
        out_specs=pl.BlockSpec((_SP, 128), lambda g, h: (h * _G + g, 0)),
        out_shape=jax.ShapeDtypeStruct((2 * _NP, 128), jnp.float32),
    )(P2v, P2v, dinvP, b1.reshape(1, _HID), W2a, gb.reshape(_G, 1, _HID))


def _t4_body(pa_ref, pb_ref, dinv_ref, b_ref, w3_ref, out_ref):
    agg = dinv_ref[...] * jnp.concatenate([pa_ref[...], pb_ref[...]], axis=1)
    h2 = jnp.maximum(agg + b_ref[...], 0.0)
    y4 = jnp.dot(h2, w3_ref[...], preferred_element_type=jnp.float32)
    out_ref[...] = y4 * dinv_ref[...]


def _t4(P3v, dinvP, b2, W3):
    return pl.pallas_call(
        _t4_body,
        grid=(_G,),
        in_specs=[pl.BlockSpec((_SP, 128), lambda g: (g, 0)),
                  pl.BlockSpec((_SP, 128), lambda g: (_G + g, 0)),
                  _rs(), _fs((1, _HID)), _fs((_HID, 1))],
        out_specs=_rs(),
        out_shape=jax.ShapeDtypeStruct((_NP, 1), jnp.float32),
    )(P3v, P3v, dinvP, b2.reshape(1, _HID), W3)


def _t5_body(pa_ref, pb_ref, y4_ref, dinv_ref, b3_ref, gum_ref, x_ref,
             xm1_ref):
    lg = (dinv_ref[...] * (pa_ref[...] + pb_ref[...] + y4_ref[...])
          + b3_ref[...])
    noisy = gum_ref[...] + lg
    rows = jax.lax.broadcasted_iota(jnp.int32, noisy.shape, 0)
    noisy = jnp.where(rows < _S, noisy, -1e30)
    m = jnp.max(noisy, axis=0, keepdims=True)
    ex = jnp.exp(noisy - m)
    s = jnp.sum(ex, axis=0, keepdims=True)
    r = ex / s
    cols = jax.lax.broadcasted_iota(jnp.int32, r.shape, 1)
    r = jnp.where(cols < 30, r, -1.0)
    t = jnp.max(r, axis=1, keepdims=True)
    xm1_ref[...] = x_ref[...] * (t * dinv_ref[...])


def _t5(P4v, y4, dinvP, b3, gumP, xP):
    return pl.pallas_call(
        _t5_body,
        grid=(_G,),
        in_specs=[pl.BlockSpec((_SP, 1), lambda g: (g, 0)),
                  pl.BlockSpec((_SP, 1), lambda g: (_G + g, 0)),
                  _rs(), _rs(), _fs((1, 1)), _rs(32), _rs(128)],
        out_specs=_rs(128),
        out_shape=jax.ShapeDtypeStruct((_NP, 128), jnp.float32),
    )(P4v, P4v, y4, dinvP, b3.reshape(1, 1), gumP, xP)


def _t6_body(pa_ref, pb_ref, xm1_ref, dinv_ref, wq1_ref, bq1_ref, p_ref,
             q1_ref, sc_ref):
    agg = dinv_ref[...] * (pa_ref[...] + pb_ref[...] - xm1_ref[...])
    q1 = jnp.dot(agg, wq1_ref[...], preferred_element_type=jnp.float32
                 ) + bq1_ref[...]
    q1_ref[...] = q1
    pv = p_ref[...]
    pn = jnp.sqrt(jnp.sum(pv * pv))
    sc_ref[...] = jnp.sum(q1 * pv, axis=1, keepdims=True) / pn


def _t6(P5v, xm1, dinvP, Wq1P, bq1P, pP):
    return pl.pallas_call(
        _t6_body,
        grid=(_G,),
        in_specs=[pl.BlockSpec((_SP, 128), lambda g: (g, 0)),
                  pl.BlockSpec((_SP, 128), lambda g: (_G + g, 0)),
                  _rs(128), _rs(),
                  _fs((128, _HID)), _fs((1, _HID)), _fs((1, _HID))],
        out_specs=[_rs(_HID), _rs()],
        out_shape=[jax.ShapeDtypeStruct((_NP, _HID), jnp.float32),
                   jax.ShapeDtypeStruct((_NP, 1), jnp.float32)],
    )(P5v, P5v, xm1, dinvP, Wq1P, bq1P.reshape(1, _HID), pP.reshape(1, _HID))


def _t7_body(sc_ref, m_ref):
    s = sc_ref[...]
    rows = jax.lax.broadcasted_iota(jnp.int32, s.shape, 0)
    valid = rows < _S
    b = jax.lax.bitcast_convert_type(s, jnp.uint32)
    key = jnp.where(b >= jnp.uint32(0x80000000), ~b,
                    b | jnp.uint32(0x80000000))
    key = jnp.where(valid, key, jnp.uint32(0))

    def kb(_, carry):
        lo, hi = carry
        mid = lo + ((hi - lo) // jnp.uint32(2)) + ((hi - lo) % jnp.uint32(2))
        cnt = jnp.sum(jnp.where(key >= mid, 1, 0))
        big = cnt >= _KP
        return (jnp.where(big, mid, lo), jnp.where(big, hi, mid - 1))

    lo, _ = jax.lax.fori_loop(0, 32, kb, (jnp.uint32(0),
                                          jnp.uint32(0xFFFFFFFF)))
    kth = lo
    c_gt = jnp.sum(jnp.where(key > kth, 1, 0))
    r = _KP - c_gt
    tie = (key == kth) & valid

    def jb(_, carry):
        lo, hi = carry
        mid = (lo + hi + 1) // 2
        cnt = jnp.sum(jnp.where(tie & (rows <= mid), 1, 0))
        ok = cnt <= r
        return (jnp.where(ok, mid, lo), jnp.where(ok, hi, mid - 1))

    jlo, _ = jax.lax.fori_loop(0, 12, jb, (jnp.int32(-1), jnp.int32(_SP - 1)))
    keep = (key > kth) | (tie & (rows <= jlo))
    m_ref[...] = jnp.where(keep, 1.0, 0.0)


def _t7(score):
    return pl.pallas_call(
        _t7_body,
        grid=(_G,),
        in_specs=[_rs()],
        out_specs=_rs(),
        out_shape=jax.ShapeDtypeStruct((_NP, 1), jnp.float32),
    )(score)


def _t8_body(q1_ref, sc_ref, m_ref, da_ref, db_ref, wq2_ref, h2p_ref):
    xp = jnp.maximum(q1_ref[...] * jnp.tanh(sc_ref[...]), 0.0)
    hfull = jnp.dot(xp, wq2_ref[...], preferred_element_type=jnp.float32)
    m = m_ref[...]
    deg2 = 1.0 + m * (da_ref[...] + db_ref[...])
    w = m * jax.lax.rsqrt(deg2)
    h2p_ref[...] = jnp.concatenate(
        [hfull * w, jnp.zeros((_SP, 96), jnp.float32)], axis=1)


def _t8(q1, score, mP, D2v, Wq2P):
    return pl.pallas_call(
        _t8_body,
        grid=(_G,),
        in_specs=[_rs(_HID), _rs(), _rs(),
                  pl.BlockSpec((_SP, 1), lambda g: (g, 0)),
                  pl.BlockSpec((_SP, 1), lambda g: (_G + g, 0)),
                  _fs((_HID, 32))],
        out_specs=_rs(128),
        out_shape=jax.ShapeDtypeStruct((_NP, 128), jnp.float32),
    )(q1, score, mP, D2v, D2v, Wq2P)


def _t9_body(pa_ref, pb_ref, h2p_ref, m_ref, da_ref, db_ref, bq2_ref,
             pool_ref):
    m = m_ref[...]
    deg2 = 1.0 + m * (da_ref[...] + db_ref[...])
    w = m * jax.lax.rsqrt(deg2)
    p6 = (pa_ref[...] + pb_ref[...] - h2p_ref[...])[:, :32]
    q2 = jnp.maximum(w * p6 + bq2_ref[...], 0.0)
    pool_ref[...] = (jnp.sum(m * q2, axis=0, keepdims=True)
                     * (1.0 / _KP))[None]


def _t9(P6v, h2p, mP, D2v, bq2):
    return pl.pallas_call(
        _t9_body,
        grid=(_G,),
        in_specs=[pl.BlockSpec((_SP, 128), lambda g: (g, 0)),
                  pl.BlockSpec((_SP, 128), lambda g: (_G + g, 0)),
                  _rs(128), _rs(),
                  pl.BlockSpec((_SP, 1), lambda g: (g, 0)),
                  pl.BlockSpec((_SP, 1), lambda g: (_G + g, 0)),
                  _fs((1, 32))],
        out_specs=pl.BlockSpec((1, 1, 32), lambda g: (g, 0, 0)),
        out_shape=jax.ShapeDtypeStruct((_G, 1, 32), jnp.float32),
    )(P6v, P6v, h2p, mP, D2v, D2v, bq2.reshape(1, 32)).reshape(_G, 32)


def _t10_body(pool_ref, wlin_ref, blin_ref, out_ref):
    out_ref[...] = jnp.dot(pool_ref[...], wlin_ref[...],
                           preferred_element_type=jnp.float32) + blin_ref[...]


def _t10(pooled, Wlin, blin):
    nc = Wlin.shape[1]
    return pl.pallas_call(
        _t10_body,
        grid=(1,),
        in_specs=[_fs((_G, 32)), _fs((32, nc)), _fs((1, nc))],
        out_specs=_fs((_G, nc)),
        out_shape=jax.ShapeDtypeStruct((_G, nc), jnp.float32),
    )(pooled, Wlin, blin.reshape(1, nc))


def kernel(x, edge_index, W0, b0, Wfc, bfc, W1, b1, W2, b2, W3, b3,
           Wq1, bq1, p, Wq2, bq2, Wlin, blin):
    src = edge_index[0]
    dst = edge_index[1]
    srcP = src + 30 * (src // _S)
    dstP = dst + 30 * (dst // _S)
    srcC = _pad_tiles(srcP, _NTC, _EPTC)
    dstC = _pad_tiles(dstP, _NTC, _EPTC)
    src2C = jnp.concatenate([srcC, srcC + _NP])
    srcE = _pad_tiles(srcP, 2 * _NTC, _EPTE)
    dstE = _pad_tiles(dstP, 2 * _NTC, _EPTE)
    pad_e = jnp.full((_EPAD - _E,), _S, jnp.int32)
    srcPp = jnp.concatenate([srcP, pad_e])
    dstPp = jnp.concatenate([dstP, pad_e])
    onesN = jnp.ones((_NP,), jnp.float32)

    xP = _pad_rows(x)
    Wq1P = jnp.pad(Wq1, ((0, 0), (0, _HID - 250)))
    bq1P = jnp.pad(bq1, (0, _HID - 250))
    pP = jnp.pad(p, (0, _HID - 250))
    Wq2P = jnp.pad(Wq2, ((0, _HID - 250), (0, 0)))
    W2a = W2[:_HID]
    W2b = W2[_HID:]

    u = jax.random.uniform(jax.random.key(42), (_N, 30), jnp.float32,
                           1e-6, 1.0 - 1e-6)
    gum = -jnp.log(-jnp.log(u))
    gumP = jnp.pad(gum.reshape(_G, _S, 30),
                   ((0, 0), (0, _SP - _S), (0, 2))).reshape(_NP, 32)

    degV = _sc_agg_scalar(onesN, srcPp, dstPp).reshape(2 * _NP, 1)

    dinvP, x1 = _t0(degV, xP)
    P0v = _sc_agg_edges(x1, srcE, dstE)
    h0, y1v = _t0b(P0v, x1, dinvP, W0, b0, W1)
    gb = _glob(h0, Wfc, bfc, W2b)
    P1v = _sc_agg_cols(y1v, src2C, dstC)
    y2v = _mid(P1v, dinvP, b1, W1)
    P2v = _sc_agg_cols(y2v, src2C, dstC)
    y3v = _t3(P2v, dinvP, b1, W2a, gb)
    P3v = _sc_agg_cols(y3v, src2C, dstC)
    y4 = _t4(P3v, dinvP, b2, W3)
    P4v = _sc_agg_scalar(y4.reshape(_NP), srcPp, dstPp).reshape(2 * _NP, 1)
    xm1 = _t5(P4v, y4, dinvP, b3, gumP, xP)
    P5v = _sc_agg_edges(xm1, srcE, dstE)
    q1, score = _t6(P5v, xm1, dinvP, Wq1P, bq1P, pP)
    mP = _t7(score)
    D2v = _sc_agg_scalar(mP.reshape(_NP), srcPp, dstPp).reshape(2 * _NP, 1)
    h2p = _t8(q1, score, mP, D2v, Wq2P)
    P6v = _sc_agg_edges(h2p, srcE, dstE)
    pooled = _t9(P6v, h2p, mP, D2v, bq2)
    return _t10(pooled, Wlin, blin)


# speedup vs baseline: 1.3765x; 1.3765x over previous
import functools

import jax
import jax.numpy as jnp
from jax import lax
from jax.experimental import pallas as pl
from jax.experimental.pallas import tpu as pltpu
from jax.experimental.pallas import tpu_sc as plsc

_N = 10000
_G = 8
_S = _N // _G
_SP = 1280
_NP = _G * _SP
_E = 160000
_HID = 256
_KP = _S // 2

_NTC = 16
_RPT = _NP // _NTC


def _pad_rows(a):
    f = a.shape[1]
    return jnp.pad(a.reshape(_G, _S, f), ((0, 0), (0, _SP - _S), (0, 0))
                   ).reshape(_NP, f)


def _sc_agg_stream(ys, src, dst, nchunks, ch, src_off, edge_base, init_off):
    mesh = plsc.VectorSubcoreMesh(core_axis_name="c", subcore_axis_name="s")

    @functools.partial(
        pl.kernel,
        out_type=jax.ShapeDtypeStruct((2 * _NP, 128), jnp.float32),
        mesh=mesh,
        scratch_types=[
            pltpu.VMEM((ch,), jnp.int32),
            pltpu.VMEM((ch,), jnp.int32),
            pltpu.VMEM((ch, 128), jnp.float32),
            pltpu.VMEM_SHARED((_NP, 128), jnp.float32),
            pltpu.SemaphoreType.DMA,
        ],
    )
    def k(ys_ref, src_ref, dst_ref, out_ref, sidx, didx, rows, acc, sem):
        c = lax.axis_index("c")
        s = lax.axis_index("s")
        r0 = s * _RPT
        pltpu.sync_copy(ys_ref.at[pl.ds(init_off(c) + r0, _RPT)],
                        acc.at[pl.ds(r0, _RPT)])
        plsc.subcore_barrier()
        base = edge_base(c, s)
        soff = src_off(c)

        def body(j, carry):
            e0 = base + j * ch
            pltpu.sync_copy(src_ref.at[pl.ds(soff + e0, ch)], sidx)
            pltpu.sync_copy(dst_ref.at[pl.ds(e0, ch)], didx)
            pltpu.async_copy(ys_ref.at[sidx], rows, sem).wait()
            pltpu.sync_copy(rows, acc.at[didx], add=True)
            return carry

        lax.fori_loop(0, nchunks, body, 0)
        plsc.subcore_barrier()
        pltpu.sync_copy(acc.at[pl.ds(r0, _RPT)],
                        out_ref.at[pl.ds(c * _NP + r0, _RPT)])

    return k(ys, src, dst)


def _sc_agg_cols(ys2, src2, dst):
    ept = _E // _NTC
    return _sc_agg_stream(
        ys2, src2, dst, nchunks=ept // 200, ch=200,
        src_off=lambda c: c * _E,
        edge_base=lambda c, s: s * ept,
        init_off=lambda c: c * _NP)


def _sc_agg_edges(ys, srcP, dstP):
    ept = _E // 2 // _NTC
    return _sc_agg_stream(
        ys, srcP, dstP, nchunks=ept // 200, ch=200,
        src_off=lambda c: 0,
        edge_base=lambda c, s: c * (_E // 2) + s * ept,
        init_off=lambda c: 0)


_EPAD = 160256
_EPT1 = _EPAD // 32


def _sc_agg_scalar(yv, srcPp, dstPp):
    mesh = plsc.VectorSubcoreMesh(core_axis_name="c", subcore_axis_name="s")

    @functools.partial(
        pl.kernel,
        out_type=jax.ShapeDtypeStruct((2 * _NP,), jnp.float32),
        mesh=mesh,
        compiler_params=pltpu.CompilerParams(needs_layout_passes=False),
        scratch_types=[
            pltpu.VMEM((_EPT1,), jnp.int32),
            pltpu.VMEM((_EPT1,), jnp.int32),
            pltpu.VMEM((_NP,), jnp.float32),
            pltpu.VMEM((_NP,), jnp.float32),
            pltpu.VMEM((_NTC, _RPT), jnp.float32),
            pltpu.VMEM((_RPT,), jnp.float32),
            pltpu.VMEM_SHARED((_NTC, _NP), jnp.float32),
            pltpu.SemaphoreType.DMA,
        ],
    )
    def k(y_ref, src_ref, dst_ref, out_ref, sidx, didx, yv_, acc, red, res,
          shared, sem):
        c = lax.axis_index("c")
        s = lax.axis_index("s")
        w = c * _NTC + s
        pltpu.sync_copy(y_ref, yv_)
        pltpu.sync_copy(src_ref.at[pl.ds(w * _EPT1, _EPT1)], sidx)
        pltpu.sync_copy(dst_ref.at[pl.ds(w * _EPT1, _EPT1)], didx)

        def zbody(i, carry):
            acc[pl.ds(i * 16, 16)] = jnp.zeros((16,), jnp.float32)
            return carry

        lax.fori_loop(0, _NP // 16, zbody, 0)

        def ebody(i, carry):
            s16 = sidx[pl.ds(i * 16, 16)]
            d16 = didx[pl.ds(i * 16, 16)]
            vals = plsc.load_gather(yv_, [s16])
            plsc.addupdate_scatter(acc, [d16], vals)
            return carry

        lax.fori_loop(0, _EPT1 // 16, ebody, 0)
        pltpu.sync_copy(acc, shared.at[s])
        plsc.subcore_barrier()
        lo = s * _RPT
        pltpu.sync_copy(shared.at[:, pl.ds(lo, _RPT)], red)

        def rbody(j, carry):
            v = red[0, pl.ds(j * 16, 16)]
            for t in range(1, _NTC):
                v = v + red[t, pl.ds(j * 16, 16)]
            res[pl.ds(j * 16, 16)] = v
            return carry

        lax.fori_loop(0, _RPT // 16, rbody, 0)
        pltpu.sync_copy(res, out_ref.at[pl.ds(c * _NP + lo, _RPT)])

    return k(yv, srcPp, dstPp)


def _rs(f=1):
    return pl.BlockSpec((_SP, f), lambda g: (g, 0))


def _rs2(f=1):
    return pl.BlockSpec((_SP, f), lambda g, h: (g, 0))


def _fs(shape):
    nd = len(shape)
    return pl.BlockSpec(shape, lambda g: (0,) * nd)


def _fs2(shape):
    nd = len(shape)
    return pl.BlockSpec(shape, lambda g, h: (0,) * nd)


def _t0_body(da_ref, db_ref, x_ref, dinv_ref, x1_ref):
    dinv = jax.lax.rsqrt(da_ref[...] + db_ref[...] + 1.0)
    dinv_ref[...] = dinv
    x1_ref[...] = x_ref[...] * dinv


def _t0(degV, xP):
    return pl.pallas_call(
        _t0_body,
        grid=(_G,),
        in_specs=[pl.BlockSpec((_SP, 1), lambda g: (g, 0)),
                  pl.BlockSpec((_SP, 1), lambda g: (_G + g, 0)),
                  _rs(128)],
        out_specs=[_rs(), _rs(128)],
        out_shape=[jax.ShapeDtypeStruct((_NP, 1), jnp.float32),
                   jax.ShapeDtypeStruct((_NP, 128), jnp.float32)],
    )(degV, degV, xP)


def _t0b_body(pa_ref, pb_ref, x1_ref, dinv_ref, w0_ref, b0_ref, w1h_ref,
              h0_ref, y1_ref):
    agg = dinv_ref[...] * (pa_ref[...] + pb_ref[...] - x1_ref[...])
    h0 = jnp.maximum(
        jnp.dot(agg, w0_ref[...], preferred_element_type=jnp.float32)
        + b0_ref[...], 0.0)
    h0_ref[...] = h0
    y1_ref[...] = jnp.dot(h0, w1h_ref[...], preferred_element_type=jnp.float32
                          ) * dinv_ref[...]


def _t0b(P0v, x1, dinvP, W0, b0, W1):
    return pl.pallas_call(
        _t0b_body,
        grid=(_G, 2),
        in_specs=[pl.BlockSpec((_SP, 128), lambda g, h: (g, 0)),
                  pl.BlockSpec((_SP, 128), lambda g, h: (_G + g, 0)),
                  _rs2(128), _rs2(),
                  _fs2((128, _HID)), _fs2((1, _HID)),
                  pl.BlockSpec((_HID, 128), lambda g, h: (0, h))],
        out_specs=[pl.BlockSpec((_SP, _HID), lambda g, h: (g, 0)),
                   pl.BlockSpec((_SP, 128), lambda g, h: (h * _G + g, 0))],
        out_shape=[jax.ShapeDtypeStruct((_NP, _HID), jnp.float32),
                   jax.ShapeDtypeStruct((2 * _NP, 128), jnp.float32)],
    )(P0v, P0v, x1, dinvP, W0, b0.reshape(1, _HID), W1)


def _glob_body(h0_ref, wfc_ref, bfc_ref, w2b_ref, gb_ref):
    h = h0_ref[...]
    rows = jax.lax.broadcasted_iota(jnp.int32, h.shape, 1)
    h = jnp.where(rows < _S, h, -1e30)
    glob = jnp.max(h, axis=1)
    gi = jnp.dot(glob, wfc_ref[...], preferred_element_type=jnp.float32
                 ) + bfc_ref[...]
    gb_ref[...] = jnp.dot(gi, w2b_ref[...], preferred_element_type=jnp.float32)


def _glob(h0, Wfc, bfc, W2b):
    return pl.pallas_call(
        _glob_body,
        grid=(1,),
        in_specs=[_fs((_G, _SP, _HID)), _fs((_HID, _HID)),
                  _fs((1, _HID)), _fs((_HID, _HID))],
        out_specs=_fs((_G, _HID)),
        out_shape=jax.ShapeDtypeStruct((_G, _HID), jnp.float32),
    )(h0.reshape(_G, _SP, _HID), Wfc, bfc.reshape(1, _HID), W2b)


def _mid_body(pa_ref, pb_ref, dinv_ref, b_ref, wh_ref, out_ref):
    agg = dinv_ref[...] * jnp.concatenate([pa_ref[...], pb_ref[...]], axis=1)
    h = jnp.maximum(agg + b_ref[...], 0.0)
    out_ref[...] = jnp.dot(h, wh_ref[...], preferred_element_type=jnp.float32
                           ) * dinv_ref[...]


def _mid(Pv, dinvP, b, W):
    return pl.pallas_call(
        _mid_body,
        grid=(_G, 2),
        in_specs=[pl.BlockSpec((_SP, 128), lambda g, h: (g, 0)),
                  pl.BlockSpec((_SP, 128), lambda g, h: (_G + g, 0)),
                  _rs2(), _fs2((1, _HID)),
                  pl.BlockSpec((_HID, 128), lambda g, h: (0, h))],
        out_specs=pl.BlockSpec((_SP, 128), lambda g, h: (h * _G + g, 0)),
        out_shape=jax.ShapeDtypeStruct((2 * _NP, 128), jnp.float32),
    )(Pv, Pv, dinvP, b.reshape(1, _HID), W)


def _t3_body(pa_ref, pb_ref, dinv_ref, b_ref, wh_ref, gb_ref, out_ref):
    agg = dinv_ref[...] * jnp.concatenate([pa_ref[...], pb_ref[...]], axis=1)
    h = jnp.maximum(agg + b_ref[...], 0.0)
    y3 = jnp.dot(h, wh_ref[...], preferred_element_type=jnp.float32
                 ) + gb_ref[0]
    out_ref[...] = y3 * dinv_ref[...]


def _t3(P2v, dinvP, b1, W2a, gb):
    return pl.pallas_call(
        _t3_body,
        grid=(_G, 2),
        in_specs=[pl.BlockSpec((_SP, 128), lambda g, h: (g, 0)),
                  pl.BlockSpec((_SP, 128), lambda g, h: (_G + g, 0)),
                  _rs2(), _fs2((1, _HID)),
                  pl.BlockSpec((_HID, 128), lambda g, h: (0, h)),
                  pl.BlockSpec((1, 1, 128), lambda g, h: (g, 0, h))],
        out_specs=pl.BlockSpec((_SP, 128), lambda g, h: (h * _G + g, 0)),
        out_shape=jax.ShapeDtypeStruct((2 * _NP, 128), jnp.float32),
    )(P2v, P2v, dinvP, b1.reshape(1, _HID), W2a, gb.reshape(_G, 1, _HID))


def _t4_body(pa_ref, pb_ref, dinv_ref, b_ref, w3_ref, out_ref):
    agg = dinv_ref[...] * jnp.concatenate([pa_ref[...], pb_ref[...]], axis=1)
    h2 = jnp.maximum(agg + b_ref[...], 0.0)
    y4 = jnp.dot(h2, w3_ref[...], preferred_element_type=jnp.float32)
    out_ref[...] = y4 * dinv_ref[...]


def _t4(P3v, dinvP, b2, W3):
    return pl.pallas_call(
        _t4_body,
        grid=(_G,),
        in_specs=[pl.BlockSpec((_SP, 128), lambda g: (g, 0)),
                  pl.BlockSpec((_SP, 128), lambda g: (_G + g, 0)),
                  _rs(), _fs((1, _HID)), _fs((_HID, 1))],
        out_specs=_rs(),
        out_shape=jax.ShapeDtypeStruct((_NP, 1), jnp.float32),
    )(P3v, P3v, dinvP, b2.reshape(1, _HID), W3)


def _t5_body(pa_ref, pb_ref, y4_ref, dinv_ref, b3_ref, gum_ref, x_ref,
             xm1_ref):
    lg = (dinv_ref[...] * (pa_ref[...] + pb_ref[...] + y4_ref[...])
          + b3_ref[...])
    noisy = gum_ref[...] + lg
    rows = jax.lax.broadcasted_iota(jnp.int32, noisy.shape, 0)
    noisy = jnp.where(rows < _S, noisy, -1e30)
    m = jnp.max(noisy, axis=0, keepdims=True)
    ex = jnp.exp(noisy - m)
    s = jnp.sum(ex, axis=0, keepdims=True)
    r = ex / s
    cols = jax.lax.broadcasted_iota(jnp.int32, r.shape, 1)
    r = jnp.where(cols < 30, r, -1.0)
    t = jnp.max(r, axis=1, keepdims=True)
    xm1_ref[...] = x_ref[...] * (t * dinv_ref[...])


def _t5(P4v, y4, dinvP, b3, gumP, xP):
    return pl.pallas_call(
        _t5_body,
        grid=(_G,),
        in_specs=[pl.BlockSpec((_SP, 1), lambda g: (g, 0)),
                  pl.BlockSpec((_SP, 1), lambda g: (_G + g, 0)),
                  _rs(), _rs(), _fs((1, 1)), _rs(32), _rs(128)],
        out_specs=_rs(128),
        out_shape=jax.ShapeDtypeStruct((_NP, 128), jnp.float32),
    )(P4v, P4v, y4, dinvP, b3.reshape(1, 1), gumP, xP)


def _t6_body(pa_ref, pb_ref, xm1_ref, dinv_ref, wq1_ref, bq1_ref, p_ref,
             q1_ref, sc_ref):
    agg = dinv_ref[...] * (pa_ref[...] + pb_ref[...] - xm1_ref[...])
    q1 = jnp.dot(agg, wq1_ref[...], preferred_element_type=jnp.float32
                 ) + bq1_ref[...]
    q1_ref[...] = q1
    pv = p_ref[...]
    pn = jnp.sqrt(jnp.sum(pv * pv))
    sc_ref[...] = jnp.sum(q1 * pv, axis=1, keepdims=True) / pn


def _t6(P5v, xm1, dinvP, Wq1P, bq1P, pP):
    return pl.pallas_call(
        _t6_body,
        grid=(_G,),
        in_specs=[pl.BlockSpec((_SP, 128), lambda g: (g, 0)),
                  pl.BlockSpec((_SP, 128), lambda g: (_G + g, 0)),
                  _rs(128), _rs(),
                  _fs((128, _HID)), _fs((1, _HID)), _fs((1, _HID))],
        out_specs=[_rs(_HID), _rs()],
        out_shape=[jax.ShapeDtypeStruct((_NP, _HID), jnp.float32),
                   jax.ShapeDtypeStruct((_NP, 1), jnp.float32)],
    )(P5v, P5v, xm1, dinvP, Wq1P, bq1P.reshape(1, _HID), pP.reshape(1, _HID))


def _t7_body(sc_ref, m_ref):
    s = sc_ref[...]
    rows = jax.lax.broadcasted_iota(jnp.int32, s.shape, 0)
    valid = rows < _S
    b = jax.lax.bitcast_convert_type(s, jnp.uint32)
    key = jnp.where(b >= jnp.uint32(0x80000000), ~b,
                    b | jnp.uint32(0x80000000))
    key = jnp.where(valid, key, jnp.uint32(0))

    def kb(_, carry):
        lo, hi = carry
        mid = lo + ((hi - lo) // jnp.uint32(2)) + ((hi - lo) % jnp.uint32(2))
        cnt = jnp.sum(jnp.where(key >= mid, 1, 0))
        big = cnt >= _KP
        return (jnp.where(big, mid, lo), jnp.where(big, hi, mid - 1))

    lo, _ = jax.lax.fori_loop(0, 32, kb, (jnp.uint32(0),
                                          jnp.uint32(0xFFFFFFFF)))
    kth = lo
    c_gt = jnp.sum(jnp.where(key > kth, 1, 0))
    r = _KP - c_gt
    tie = (key == kth) & valid

    def jb(_, carry):
        lo, hi = carry
        mid = (lo + hi + 1) // 2
        cnt = jnp.sum(jnp.where(tie & (rows <= mid), 1, 0))
        ok = cnt <= r
        return (jnp.where(ok, mid, lo), jnp.where(ok, hi, mid - 1))

    jlo, _ = jax.lax.fori_loop(0, 12, jb, (jnp.int32(-1), jnp.int32(_SP - 1)))
    keep = (key > kth) | (tie & (rows <= jlo))
    m_ref[...] = jnp.where(keep, 1.0, 0.0)


def _t7(score):
    return pl.pallas_call(
        _t7_body,
        grid=(_G,),
        in_specs=[_rs()],
        out_specs=_rs(),
        out_shape=jax.ShapeDtypeStruct((_NP, 1), jnp.float32),
    )(score)


def _t8_body(q1_ref, sc_ref, m_ref, da_ref, db_ref, wq2_ref, h2p_ref):
    xp = jnp.maximum(q1_ref[...] * jnp.tanh(sc_ref[...]), 0.0)
    hfull = jnp.dot(xp, wq2_ref[...], preferred_element_type=jnp.float32)
    m = m_ref[...]
    deg2 = 1.0 + m * (da_ref[...] + db_ref[...])
    w = m * jax.lax.rsqrt(deg2)
    h2p_ref[...] = jnp.concatenate(
        [hfull * w, jnp.zeros((_SP, 96), jnp.float32)], axis=1)


def _t8(q1, score, mP, D2v, Wq2P):
    return pl.pallas_call(
        _t8_body,
        grid=(_G,),
        in_specs=[_rs(_HID), _rs(), _rs(),
                  pl.BlockSpec((_SP, 1), lambda g: (g, 0)),
                  pl.BlockSpec((_SP, 1), lambda g: (_G + g, 0)),
                  _fs((_HID, 32))],
        out_specs=_rs(128),
        out_shape=jax.ShapeDtypeStruct((_NP, 128), jnp.float32),
    )(q1, score, mP, D2v, D2v, Wq2P)


def _t9_body(pa_ref, pb_ref, h2p_ref, m_ref, da_ref, db_ref, bq2_ref,
             pool_ref):
    m = m_ref[...]
    deg2 = 1.0 + m * (da_ref[...] + db_ref[...])
    w = m * jax.lax.rsqrt(deg2)
    p6 = (pa_ref[...] + pb_ref[...] - h2p_ref[...])[:, :32]
    q2 = jnp.maximum(w * p6 + bq2_ref[...], 0.0)
    pool_ref[...] = (jnp.sum(m * q2, axis=0, keepdims=True)
                     * (1.0 / _KP))[None]


def _t9(P6v, h2p, mP, D2v, bq2):
    return pl.pallas_call(
        _t9_body,
        grid=(_G,),
        in_specs=[pl.BlockSpec((_SP, 128), lambda g: (g, 0)),
                  pl.BlockSpec((_SP, 128), lambda g: (_G + g, 0)),
                  _rs(128), _rs(),
                  pl.BlockSpec((_SP, 1), lambda g: (g, 0)),
                  pl.BlockSpec((_SP, 1), lambda g: (_G + g, 0)),
                  _fs((1, 32))],
        out_specs=pl.BlockSpec((1, 1, 32), lambda g: (g, 0, 0)),
        out_shape=jax.ShapeDtypeStruct((_G, 1, 32), jnp.float32),
    )(P6v, P6v, h2p, mP, D2v, D2v, bq2.reshape(1, 32)).reshape(_G, 32)


def _t10_body(pool_ref, wlin_ref, blin_ref, out_ref):
    out_ref[...] = jnp.dot(pool_ref[...], wlin_ref[...],
                           preferred_element_type=jnp.float32) + blin_ref[...]


def _t10(pooled, Wlin, blin):
    nc = Wlin.shape[1]
    return pl.pallas_call(
        _t10_body,
        grid=(1,),
        in_specs=[_fs((_G, 32)), _fs((32, nc)), _fs((1, nc))],
        out_specs=_fs((_G, nc)),
        out_shape=jax.ShapeDtypeStruct((_G, nc), jnp.float32),
    )(pooled, Wlin, blin.reshape(1, nc))


def kernel(x, edge_index, W0, b0, Wfc, bfc, W1, b1, W2, b2, W3, b3,
           Wq1, bq1, p, Wq2, bq2, Wlin, blin):
    src = edge_index[0]
    dst = edge_index[1]
    srcP = src + 30 * (src // _S)
    dstP = dst + 30 * (dst // _S)
    src2 = jnp.concatenate([srcP, srcP + _NP])
    pad_e = jnp.full((_EPAD - _E,), _S, jnp.int32)
    srcPp = jnp.concatenate([srcP, pad_e])
    dstPp = jnp.concatenate([dstP, pad_e])
    onesN = jnp.ones((_NP,), jnp.float32)

    xP = _pad_rows(x)
    Wq1P = jnp.pad(Wq1, ((0, 0), (0, _HID - 250)))
    bq1P = jnp.pad(bq1, (0, _HID - 250))
    pP = jnp.pad(p, (0, _HID - 250))
    Wq2P = jnp.pad(Wq2, ((0, _HID - 250), (0, 0)))
    W2a = W2[:_HID]
    W2b = W2[_HID:]

    u = jax.random.uniform(jax.random.key(42), (_N, 30), jnp.float32,
                           1e-6, 1.0 - 1e-6)
    gum = -jnp.log(-jnp.log(u))
    gumP = jnp.pad(gum.reshape(_G, _S, 30),
                   ((0, 0), (0, _SP - _S), (0, 2))).reshape(_NP, 32)

    degV = _sc_agg_scalar(onesN, srcPp, dstPp).reshape(2 * _NP, 1)

    dinvP, x1 = _t0(degV, xP)
    P0v = _sc_agg_edges(x1, srcP, dstP)
    h0, y1v = _t0b(P0v, x1, dinvP, W0, b0, W1)
    gb = _glob(h0, Wfc, bfc, W2b)
    P1v = _sc_agg_cols(y1v, src2, dstP)
    y2v = _mid(P1v, dinvP, b1, W1)
    P2v = _sc_agg_cols(y2v, src2, dstP)
    y3v = _t3(P2v, dinvP, b1, W2a, gb)
    P3v = _sc_agg_cols(y3v, src2, dstP)
    y4 = _t4(P3v, dinvP, b2, W3)
    P4v = _sc_agg_scalar(y4.reshape(_NP), srcPp, dstPp).reshape(2 * _NP, 1)
    xm1 = _t5(P4v, y4, dinvP, b3, gumP, xP)
    P5v = _sc_agg_edges(xm1, srcP, dstP)
    q1, score = _t6(P5v, xm1, dinvP, Wq1P, bq1P, pP)
    mP = _t7(score)
    D2v = _sc_agg_scalar(mP.reshape(_NP), srcPp, dstPp).reshape(2 * _NP, 1)
    h2p = _t8(q1, score, mP, D2v, Wq2P)
    P6v = _sc_agg_edges(h2p, srcP, dstP)
    pooled = _t9(P6v, h2p, mP, D2v, bq2)
    return _t10(pooled, Wlin, blin)
